# baseline (speedup 1.0000x reference)
import functools

import jax
import jax.numpy as jnp
from jax import lax
from jax.experimental import pallas as pl
from jax.experimental.pallas import tpu as pltpu
from jax.experimental.pallas import tpu_sc as plsc

BATCH = 16384
DIM = 512
L = 16

NC = 2
NS = 16
NW = NC * NS

SC_ROWS = 8192
TC_ROWS = BATCH - SC_ROWS
BPW = SC_ROWS // NW
CHUNK = 64
NCHUNK = BPW // CHUNK
NBUF = 3

TC_BLK = 2048

_mesh = plsc.VectorSubcoreMesh(
    core_axis_name="c", subcore_axis_name="s", num_cores=NC, num_subcores=NS)


@functools.partial(
    pl.kernel,
    out_type=jax.ShapeDtypeStruct((SC_ROWS, DIM), jnp.float32),
    mesh=_mesh,
    scratch_types=[
        pltpu.VMEM((BPW,), jnp.int32),
        pltpu.VMEM((2, DIM), jnp.float32),
        pltpu.VMEM((NBUF, CHUNK, DIM), jnp.float32),
        pltpu.SemaphoreType.DMA,
        pltpu.SemaphoreType.DMA,
        pltpu.SemaphoreType.DMA,
    ],
)
def _embed_sc(cond_hbm, w_hbm, out_hbm, cond_v, wv, rows_v, ws0, ws1, ws2):
    wid = lax.axis_index("s") * NC + lax.axis_index("c")
    base = wid * BPW
    wsems = [ws0, ws1, ws2]

    pltpu.sync_copy(cond_hbm.at[pl.ds(base, BPW)], cond_v)
    pltpu.sync_copy(w_hbm, wv)

    wh = [None] * NBUF
    for ch in range(NCHUNK):
        p = ch % NBUF
        if wh[p] is not None:
            wh[p].wait()
        for g in range(CHUNK // L):
            cvec = cond_v[pl.ds(ch * CHUNK + g * L, L)].astype(jnp.float32)
            cf = [
                cvec.at[jnp.full((L,), r, jnp.int32)].get(
                    mode="promise_in_bounds")
                for r in range(L)
            ]
            cg = [1.0 - cf[r] for r in range(L)]

            @plsc.parallel_loop(0, DIM // L, step=1, unroll=2)
            def _col(j, _g=g, _p=p, _cf=cf, _cg=cg):
                w0j = wv[0, pl.ds(j * L, L)]
                w1j = wv[1, pl.ds(j * L, L)]
                for r in range(L):
                    rows_v[_p, _g * L + r, pl.ds(j * L, L)] = (
                        _cf[r] * w1j + _cg[r] * w0j)

        wh[p] = pltpu.async_copy(
            rows_v.at[p], out_hbm.at[pl.ds(base + ch * CHUNK, CHUNK)],
            wsems[p])
    for ch in range(max(0, NCHUNK - NBUF), NCHUNK):
        wh[ch % NBUF].wait()


def _tc_body(cond_ref, w_ref, out_ref):
    c = cond_ref[0, :]
    out_ref[...] = jnp.where(c[:, None] > 0, w_ref[1][None, :],
                             w_ref[0][None, :])


_embed_tc = pl.pallas_call(
    _tc_body,
    grid=(TC_ROWS // TC_BLK,),
    in_specs=[
        pl.BlockSpec((None, 1, TC_BLK), lambda i: (i, 0, 0)),
        pl.BlockSpec((2, DIM), lambda i: (0, 0)),
    ],
    out_specs=pl.BlockSpec((TC_BLK, DIM), lambda i: (i, 0)),
    out_shape=jax.ShapeDtypeStruct((TC_ROWS, DIM), jnp.float32),
)


def kernel(condition, embed_weight):
    sc_out = _embed_sc(condition[:SC_ROWS], embed_weight)
    cond_tc = condition[SC_ROWS:].reshape(TC_ROWS // TC_BLK, 1, TC_BLK)
    tc_out = _embed_tc(cond_tc, embed_weight)
    return jnp.concatenate([sc_out, tc_out], axis=0)


# speedup vs baseline: 1.4621x; 1.4621x over previous
import functools

import jax
import jax.numpy as jnp
from jax import lax
from jax.experimental import pallas as pl
from jax.experimental.pallas import tpu as pltpu
from jax.experimental.pallas import tpu_sc as plsc

BATCH = 16384
DIM = 512
L = 16

NC = 2
NS = 16
NW = NC * NS
BPW = BATCH // NW
CHUNK = 32
NCHUNK = BPW // CHUNK
NBUF = 6

_mesh = plsc.VectorSubcoreMesh(
    core_axis_name="c", subcore_axis_name="s", num_cores=NC, num_subcores=NS)


@functools.partial(
    pl.kernel,
    out_type=jax.ShapeDtypeStruct((BATCH, DIM), jnp.float32),
    mesh=_mesh,
    scratch_types=[
        pltpu.VMEM((BPW,), jnp.int32),
        pltpu.VMEM((2, DIM), jnp.float32),
        pltpu.VMEM((NBUF, CHUNK, DIM), jnp.float32),
        pltpu.SemaphoreType.DMA,
        pltpu.SemaphoreType.DMA,
        pltpu.SemaphoreType.DMA,
        pltpu.SemaphoreType.DMA,
        pltpu.SemaphoreType.DMA,
        pltpu.SemaphoreType.DMA,
    ],
)
def _embed_sc(cond_hbm, w_hbm, out_hbm, cond_v, wv, rows_v,
              ws0, ws1, ws2, ws3, ws4, ws5):
    wid = lax.axis_index("s") * NC + lax.axis_index("c")
    base = wid * BPW
    wsems = [ws0, ws1, ws2, ws3, ws4, ws5]

    pltpu.sync_copy(cond_hbm.at[pl.ds(base, BPW)], cond_v)
    pltpu.sync_copy(w_hbm, wv)

    wh = [None] * NBUF
    for ch in range(NCHUNK):
        p = ch % NBUF
        if wh[p] is not None:
            wh[p].wait()
        for g in range(CHUNK // L):
            cvec = cond_v[pl.ds(ch * CHUNK + g * L, L)].astype(jnp.float32)
            cf = [
                cvec.at[jnp.full((L,), r, jnp.int32)].get(
                    mode="promise_in_bounds")
                for r in range(L)
            ]
            cg = [1.0 - cf[r] for r in range(L)]

            @plsc.parallel_loop(0, DIM // L, step=1, unroll=2)
            def _col(j, _g=g, _p=p, _cf=cf, _cg=cg):
                w0j = wv[0, pl.ds(j * L, L)]
                w1j = wv[1, pl.ds(j * L, L)]
                for r in range(L):
                    rows_v[_p, _g * L + r, pl.ds(j * L, L)] = (
                        _cf[r] * w1j + _cg[r] * w0j)

        wh[p] = pltpu.async_copy(
            rows_v.at[p], out_hbm.at[pl.ds(base + ch * CHUNK, CHUNK)],
            wsems[p])
    for ch in range(max(0, NCHUNK - NBUF), NCHUNK):
        wh[ch % NBUF].wait()


def kernel(condition, embed_weight):
    return _embed_sc(condition, embed_weight)


# speedup vs baseline: 1.7273x; 1.1814x over previous
import functools

import jax
import jax.numpy as jnp
from jax import lax
from jax.experimental import pallas as pl
from jax.experimental.pallas import tpu as pltpu
from jax.experimental.pallas import tpu_sc as plsc

BATCH = 16384
DIM = 512
L = 16

NC = 2
NS = 16
NW = NC * NS
BPW = BATCH // NW
CHUNK = 32
NCHUNK = BPW // CHUNK
NBUF = 4

_mesh = plsc.VectorSubcoreMesh(
    core_axis_name="c", subcore_axis_name="s", num_cores=NC, num_subcores=NS)


@functools.partial(
    pl.kernel,
    out_type=jax.ShapeDtypeStruct((BATCH, DIM), jnp.float32),
    mesh=_mesh,
    scratch_types=[
        pltpu.VMEM((BPW,), jnp.int32),
        pltpu.VMEM((2, DIM), jnp.float32),
        pltpu.VMEM((NBUF, CHUNK, DIM), jnp.float32),
        pltpu.SemaphoreType.DMA((NBUF,)),
    ],
)
def _embed_sc(cond_hbm, w_hbm, out_hbm, cond_v, wv, rows_v, sems):
    wid = lax.axis_index("s") * NC + lax.axis_index("c")
    base = wid * BPW

    pltpu.sync_copy(cond_hbm.at[pl.ds(base, BPW)], cond_v)
    pltpu.sync_copy(w_hbm, wv)

    def chunk_body(ch, _):
        p = lax.rem(ch, NBUF)
        row0 = ch * CHUNK
        dst = out_hbm.at[pl.ds(base + row0, CHUNK)]

        @pl.when(ch >= NBUF)
        def _():
            pltpu.make_async_copy(rows_v.at[p], dst, sems.at[p]).wait()

        for g in range(CHUNK // L):
            cvec = cond_v[pl.ds(row0 + g * L, L)].astype(jnp.float32)
            cf = [
                cvec.at[jnp.full((L,), r, jnp.int32)].get(
                    mode="promise_in_bounds")
                for r in range(L)
            ]
            cg = [1.0 - cf[r] for r in range(L)]

            @plsc.parallel_loop(0, DIM // L, step=1, unroll=2)
            def _col(j, _g=g, _cf=cf, _cg=cg):
                w0j = wv[0, pl.ds(j * L, L)]
                w1j = wv[1, pl.ds(j * L, L)]
                for r in range(L):
                    rows_v[p, _g * L + r, pl.ds(j * L, L)] = (
                        _cf[r] * w1j + _cg[r] * w0j)

        pltpu.async_copy(rows_v.at[p], dst, sems.at[p])
        return _

    lax.fori_loop(0, NCHUNK, chunk_body, None)
    for i in range(NBUF):
        pltpu.make_async_copy(
            rows_v.at[i], out_hbm.at[pl.ds(base, CHUNK)], sems.at[i]).wait()


def kernel(condition, embed_weight):
    return _embed_sc(condition, embed_weight)
